# Initial kernel scaffold; baseline (speedup 1.0000x reference)
import jax
import jax.numpy as jnp
from jax.experimental import pallas as pl


def kernel(x, edge_index, syn_emb, lemma_emb, pos_emb, sense_emb, ln_g, ln_b, Wl1, bl1, Wr1, Wl2, bl2, Wr2):
    raise NotImplementedError("write your pallas kernel here")


# speedup vs baseline: 3.7829x; 3.7829x over previous
import functools

import jax
import jax.numpy as jnp
from jax import lax
from jax.experimental import pallas as pl
from jax.experimental.pallas import tpu as pltpu
from jax.experimental.pallas import tpu_sc as plsc

N = 10000
E = 320000
D = 128
NW = 32
NPAD = 10240
NT = NPAD // NW
NC_CH = 64
ET = E // NW
EC = 80
ENC = ET // EC
RT = NPAD // 16
CW = 16
LN_EPS = 1e-12
TBLK = 1024

_mesh = plsc.VectorSubcoreMesh(core_axis_name="c", subcore_axis_name="s")


@functools.partial(
    pl.kernel,
    out_type=jax.ShapeDtypeStruct((NPAD, D), jnp.float32),
    mesh=_mesh,
    scratch_types=[
        pltpu.VMEM((NC_CH,), jnp.int32),
        pltpu.VMEM((NC_CH,), jnp.int32),
        pltpu.VMEM((NC_CH,), jnp.int32),
        pltpu.VMEM((NC_CH,), jnp.int32),
        pltpu.VMEM((NC_CH, D), jnp.float32),
        pltpu.VMEM((NC_CH, D), jnp.float32),
        pltpu.VMEM((NC_CH, D), jnp.float32),
        pltpu.VMEM((NC_CH, D), jnp.float32),
        pltpu.SemaphoreType.DMA,
    ],
)
def _embed(i0_h, i1_h, i2_h, i3_h, t0_h, t1_h, t2_h, t3_h, emb_h,
           i0, i1, i2, i3, b0, b1, b2, b3, sem):
    cid = lax.axis_index("c")
    sid = lax.axis_index("s")
    wid = sid * 2 + cid
    base = wid * NT
    for ch in range(NT // NC_CH):
        off = base + ch * NC_CH
        pltpu.sync_copy(i0_h.at[pl.ds(off, NC_CH)], i0)
        pltpu.sync_copy(i1_h.at[pl.ds(off, NC_CH)], i1)
        pltpu.sync_copy(i2_h.at[pl.ds(off, NC_CH)], i2)
        pltpu.sync_copy(i3_h.at[pl.ds(off, NC_CH)], i3)
        c0 = pltpu.async_copy(t0_h.at[i0], b0, sem)
        c1 = pltpu.async_copy(t1_h.at[i1], b1, sem)
        c2 = pltpu.async_copy(t2_h.at[i2], b2, sem)
        c3 = pltpu.async_copy(t3_h.at[i3], b3, sem)
        c0.wait()
        c1.wait()
        c2.wait()
        c3.wait()

        def srow(r, carry):
            for j in range(D // 16):
                sl = pl.ds(j * 16, 16)
                b0[r, sl] = b0[r, sl] + b1[r, sl] + b2[r, sl] + b3[r, sl]
            return carry

        lax.fori_loop(0, NC_CH, srow, 0)
        pltpu.sync_copy(b0, emb_h.at[pl.ds(off, NC_CH)])


def _make_spmm(with_cnt):
    outs = [jax.ShapeDtypeStruct((2, NPAD, D), jnp.float32)]
    scratch = [
        pltpu.VMEM((EC,), jnp.int32),
        pltpu.VMEM((EC,), jnp.int32),
        pltpu.VMEM((EC, D), jnp.float32),
        pltpu.VMEM_SHARED((NPAD, D), jnp.float32),
        pltpu.SemaphoreType.DMA,
    ]
    if with_cnt:
        outs.append(jax.ShapeDtypeStruct((2 * NPAD,), jnp.float32))
        scratch.append(pltpu.VMEM((EC,), jnp.float32))
        scratch.append(pltpu.VMEM((RT,), jnp.float32))
        scratch.append(pltpu.VMEM_SHARED((NPAD,), jnp.float32))

    def body(src_h, dst_h, h_h, *refs):
        if with_cnt:
            agg_h, cnt_h, sidx, didx, rows, acc_sh, sem, w1, z1, cnt_sh = refs
        else:
            agg_h, sidx, didx, rows, acc_sh, sem = refs
        cid = lax.axis_index("c")
        sid = lax.axis_index("s")
        wid = sid * 2 + cid
        rbase = sid * RT

        def zrow(r, carry):
            for j in range(D // 16):
                rows[r, pl.ds(j * 16, 16)] = jnp.zeros((16,), jnp.float32)
            return carry

        lax.fori_loop(0, EC, zrow, 0)
        for j in range(RT // EC):
            pltpu.sync_copy(rows, acc_sh.at[pl.ds(rbase + j * EC, EC)])
        if with_cnt:
            def z16(r, carry):
                z1[pl.ds(r * 16, 16)] = jnp.zeros((16,), jnp.float32)
                return carry

            lax.fori_loop(0, RT // 16, z16, 0)
            pltpu.sync_copy(z1, cnt_sh.at[pl.ds(rbase, RT)])

            def o16(r, carry):
                w1[pl.ds(r * 16, 16)] = jnp.ones((16,), jnp.float32)
                return carry

            lax.fori_loop(0, EC // 16, o16, 0)
        plsc.subcore_barrier()

        ebase = wid * ET

        def echunk(i, carry):
            off = ebase + i * EC
            pltpu.sync_copy(src_h.at[pl.ds(off, EC)], sidx)
            pltpu.sync_copy(dst_h.at[pl.ds(off, EC)], didx)
            pltpu.async_copy(h_h.at[sidx], rows, sem).wait()
            pltpu.sync_copy(rows, acc_sh.at[didx], add=True)
            if with_cnt:
                pltpu.sync_copy(w1, cnt_sh.at[didx], add=True)
            return carry

        lax.fori_loop(0, ENC, echunk, 0)
        plsc.subcore_barrier()

        for j in range(RT // EC):
            r0 = rbase + j * EC
            pltpu.sync_copy(acc_sh.at[pl.ds(r0, EC)], rows)
            pltpu.sync_copy(rows, agg_h.at[cid, pl.ds(r0, EC)])
        if with_cnt:
            pltpu.sync_copy(cnt_sh.at[pl.ds(rbase, RT)], z1)
            pltpu.sync_copy(z1, cnt_h.at[pl.ds(cid * NPAD + rbase, RT)])

    return pl.kernel(
        body,
        out_type=tuple(outs) if with_cnt else outs[0],
        mesh=_mesh,
        scratch_types=scratch,
    )


_spmm_cnt = _make_spmm(True)
_spmm = _make_spmm(False)


def _ln_body(emb_ref, g_ref, b_ref, out_ref):
    e = emb_ref[...]
    mu = jnp.mean(e, axis=-1, keepdims=True)
    d = e - mu
    var = jnp.mean(d * d, axis=-1, keepdims=True)
    out_ref[...] = d * lax.rsqrt(var + LN_EPS) * g_ref[...] + b_ref[...]


_ln = pl.pallas_call(
    _ln_body,
    grid=(NPAD // TBLK,),
    in_specs=[
        pl.BlockSpec((TBLK, D), lambda i: (i, 0)),
        pl.BlockSpec((1, D), lambda i: (0, 0)),
        pl.BlockSpec((1, D), lambda i: (0, 0)),
    ],
    out_specs=pl.BlockSpec((TBLK, D), lambda i: (i, 0)),
    out_shape=jax.ShapeDtypeStruct((NPAD, D), jnp.float32),
)


def _sage_body(p_ref, cnt_ref, h_ref, wlT_ref, bl_ref, wrT_ref, out_ref, *, relu):
    p = p_ref[0] + p_ref[1]
    cnt = cnt_ref[0, :] + cnt_ref[1, :]
    mean = p * (1.0 / jnp.maximum(cnt, 1.0))[:, None]
    y = (jnp.dot(mean, wlT_ref[...], preferred_element_type=jnp.float32)
         + bl_ref[...]
         + jnp.dot(h_ref[...], wrT_ref[...], preferred_element_type=jnp.float32))
    if relu:
        y = jnp.maximum(y, 0.0)
    out_ref[...] = y


def _make_sage(relu):
    return pl.pallas_call(
        functools.partial(_sage_body, relu=relu),
        grid=(NPAD // TBLK,),
        in_specs=[
            pl.BlockSpec((2, TBLK, D), lambda i: (0, i, 0)),
            pl.BlockSpec((2, TBLK), lambda i: (0, i)),
            pl.BlockSpec((TBLK, D), lambda i: (i, 0)),
            pl.BlockSpec((D, D), lambda i: (0, 0)),
            pl.BlockSpec((1, D), lambda i: (0, 0)),
            pl.BlockSpec((D, D), lambda i: (0, 0)),
        ],
        out_specs=pl.BlockSpec((TBLK, D), lambda i: (i, 0)),
        out_shape=jax.ShapeDtypeStruct((NPAD, D), jnp.float32),
    )


_sage_relu = _make_sage(True)
_sage_lin = _make_sage(False)


def kernel(x, edge_index, syn_emb, lemma_emb, pos_emb, sense_emb, ln_g, ln_b,
           Wl1, bl1, Wr1, Wl2, bl2, Wr2):
    x = x.astype(jnp.int32)
    src = edge_index[0].astype(jnp.int32)
    dst = edge_index[1].astype(jnp.int32)
    pad = NPAD - N
    i_syn = jnp.pad(x[:, 0], (0, pad))
    i_pos = jnp.pad(x[:, 1], (0, pad))
    i_sen = jnp.pad(x[:, 2], (0, pad))
    i_lem = jnp.pad(x[:, 3], (0, pad))
    emb = _embed(i_syn, i_pos, i_sen, i_lem,
                 syn_emb, pos_emb, sense_emb, lemma_emb)
    h = _ln(emb, ln_g.reshape(1, D), ln_b.reshape(1, D))
    p1, cnt = _spmm_cnt(src, dst, h)
    cnt = cnt.reshape(2, NPAD)
    h1 = _sage_relu(p1, cnt, h, Wl1.T, bl1.reshape(1, D), Wr1.T)
    p2 = _spmm(src, dst, h1)
    out = _sage_lin(p2, cnt, h1, Wl2.T, bl2.reshape(1, D), Wr2.T)
    return out[:N]
